# Initial kernel scaffold; baseline (speedup 1.0000x reference)
import jax
import jax.numpy as jnp
from jax.experimental import pallas as pl


def kernel(h, edge_index, W_att, b_att, W_t, b_t):
    raise NotImplementedError("write your pallas kernel here")


# speedup vs baseline: 8.5943x; 8.5943x over previous
import functools

import jax
import jax.numpy as jnp
from jax import lax
from jax.experimental import pallas as pl
from jax.experimental.pallas import tpu as pltpu
from jax.experimental.pallas import tpu_sc as plsc

N = 10000
E = 320000
D = 128

NC = 2
NS = 16
NW = NC * NS
EPW = E // NW
C = 80
NCHUNK = EPW // C
RB = 80
NRB = N // RB
TPS = (NRB + NS - 1) // NS
NG = 5
CPG = NCHUNK // NG
RBLK = 2000
LG = C // 16

_mesh = plsc.VectorSubcoreMesh(core_axis_name="c", subcore_axis_name="s")
_sc_params = pltpu.CompilerParams(needs_layout_passes=False)


def _mm_body(h_ref, w_ref, o_ref):
    o_ref[...] = jnp.dot(h_ref[...], w_ref[...],
                         preferred_element_type=jnp.float32)


def _matmul(h, w_cat):
    return pl.pallas_call(
        _mm_body,
        grid=(N // RBLK,),
        in_specs=[
            pl.BlockSpec((RBLK, D), lambda i: (i, 0)),
            pl.BlockSpec((D, 2 * D), lambda i: (0, 0)),
        ],
        out_specs=pl.BlockSpec((RBLK, 2 * D), lambda i: (i, 0)),
        out_shape=jax.ShapeDtypeStruct((N, 2 * D), jnp.float32),
    )(h, w_cat)


@functools.partial(
    pl.kernel,
    out_type=jax.ShapeDtypeStruct((NW, NCHUNK, C), jnp.float32),
    mesh=_mesh,
    compiler_params=_sc_params,
    scratch_types=[
        pltpu.VMEM((N,), jnp.float32),
        pltpu.VMEM((N,), jnp.float32),
        pltpu.VMEM((NCHUNK, C), jnp.int32),
        pltpu.VMEM((NCHUNK, C), jnp.int32),
        pltpu.VMEM((NCHUNK, C), jnp.float32),
    ],
)
def _logits_k(a1_hbm, a2_hbm, src_hbm, dst_hbm, l_hbm,
              a1_v, a2_v, src_v, dst_v, l_v):
    cid = lax.axis_index("c")
    sid = lax.axis_index("s")
    wid = cid * NS + sid
    pltpu.sync_copy(a1_hbm, a1_v)
    pltpu.sync_copy(a2_hbm, a2_v)
    pltpu.sync_copy(src_hbm.at[wid], src_v)
    pltpu.sync_copy(dst_hbm.at[wid], dst_v)

    def chunk(k, carry):
        for g in range(LG):
            s_idx = src_v[k, pl.ds(g * 16, 16)]
            d_idx = dst_v[k, pl.ds(g * 16, 16)]
            v = plsc.load_gather(a1_v, [s_idx]) + plsc.load_gather(a2_v, [d_idx])
            l_v[k, pl.ds(g * 16, 16)] = jnp.maximum(v, 0.2 * v)
        return carry

    lax.fori_loop(0, NCHUNK, chunk, 0)
    pltpu.sync_copy(l_v, l_hbm.at[wid])


def _softmax_body(l_ref, o_ref):
    x = l_ref[...]
    e = jnp.exp(x - jnp.max(x))
    o_ref[...] = e * (1.0 / jnp.sum(e))


def _softmax(l2d):
    return pl.pallas_call(
        _softmax_body,
        out_shape=jax.ShapeDtypeStruct(l2d.shape, jnp.float32),
    )(l2d)


@functools.partial(
    pl.kernel,
    out_type=jax.ShapeDtypeStruct((NC, N, D), jnp.float32),
    mesh=_mesh,
    compiler_params=_sc_params,
    scratch_types=[
        pltpu.VMEM((CPG, C), jnp.int32),
        pltpu.VMEM((CPG, C), jnp.int32),
        pltpu.VMEM((CPG, C), jnp.float32),
        pltpu.VMEM((C, D), jnp.float32),
        pltpu.VMEM_SHARED((N, D), jnp.float32),
        pltpu.SemaphoreType.DMA,
    ],
)
def _scatter_k(g_hbm, src_hbm, dst_hbm, att_hbm, z_hbm,
               src_v, dst_v, att_v, rows_v, z_sh, sem):
    cid = lax.axis_index("c")
    sid = lax.axis_index("s")
    wid = cid * NS + sid

    def zrow(i, carry):
        for j in range(D // 16):
            rows_v[i, pl.ds(j * 16, 16)] = jnp.zeros((16,), jnp.float32)
        return carry

    lax.fori_loop(0, RB, zrow, 0)
    for t in range(TPS):
        b = sid + t * NS

        @pl.when(b < NRB)
        def _zero():
            off = pl.multiple_of(b * RB, RB)
            pltpu.sync_copy(rows_v, z_sh.at[pl.ds(off, RB)])

    plsc.subcore_barrier()

    def group(gi, carry):
        pltpu.sync_copy(src_hbm.at[wid, gi], src_v)
        pltpu.sync_copy(dst_hbm.at[wid, gi], dst_v)
        pltpu.sync_copy(att_hbm.at[wid, gi], att_v)

        def chunk(k, c1):
            pltpu.async_copy(g_hbm.at[src_v.at[k]], rows_v, sem).wait()

            def scale(g, c2):
                av = att_v[k, pl.ds(g * 16, 16)]
                for r in range(16):
                    s = av[r]
                    i = g * 16 + r
                    for j in range(D // 16):
                        rows_v[i, pl.ds(j * 16, 16)] = (
                            rows_v[i, pl.ds(j * 16, 16)] * s)
                return c2

            lax.fori_loop(0, LG, scale, 0)
            pltpu.sync_copy(rows_v, z_sh.at[dst_v.at[k]], add=True)
            return c1

        lax.fori_loop(0, CPG, chunk, 0)
        return carry

    lax.fori_loop(0, NG, group, 0)
    plsc.subcore_barrier()

    for t in range(TPS):
        b = sid + t * NS

        @pl.when(b < NRB)
        def _wb():
            off = pl.multiple_of(b * RB, RB)
            pltpu.sync_copy(z_sh.at[pl.ds(off, RB)], rows_v)
            pltpu.sync_copy(rows_v, z_hbm.at[cid, pl.ds(off, RB)])


def _final_body(z_ref, h_ref, b_ref, o_ref):
    z = z_ref[0] + z_ref[1] + b_ref[...]
    o_ref[...] = jnp.maximum(z, 0.0) + h_ref[...]


def _final(z, h, b_t2d):
    return pl.pallas_call(
        _final_body,
        grid=(N // RBLK,),
        in_specs=[
            pl.BlockSpec((NC, RBLK, D), lambda i: (0, i, 0)),
            pl.BlockSpec((RBLK, D), lambda i: (i, 0)),
            pl.BlockSpec((1, D), lambda i: (0, 0)),
        ],
        out_specs=pl.BlockSpec((RBLK, D), lambda i: (i, 0)),
        out_shape=jax.ShapeDtypeStruct((N, D), jnp.float32),
    )(z, h, b_t2d)


def kernel(h, edge_index, W_att, b_att, W_t, b_t):
    src = edge_index[0].astype(jnp.int32).reshape(NW, NCHUNK, C)
    dst = edge_index[1].astype(jnp.int32).reshape(NW, NCHUNK, C)

    wa1 = W_att[0, :D]
    wa2 = W_att[0, D:]
    w_cat = jnp.concatenate(
        [W_t.T, wa1[:, None], wa2[:, None],
         jnp.zeros((D, D - 2), jnp.float32)], axis=1)

    hw = _matmul(h, w_cat)
    g = hw[:, :D]
    a1 = hw[:, D] + b_att[0]
    a2 = hw[:, D + 1]

    l = _logits_k(a1, a2, src, dst)
    att2d = _softmax(l.reshape(E // D, D))
    att = att2d.reshape(E)

    z = _scatter_k(g, src.reshape(NW, NG, CPG, C), dst.reshape(NW, NG, CPG, C),
                   att.reshape(NW, NG, CPG, C))
    h_new = _final(z, h, b_t.reshape(1, D))
    return (h_new, att)
